# Initial kernel scaffold; baseline (speedup 1.0000x reference)
import jax
import jax.numpy as jnp
from jax.experimental import pallas as pl


def kernel(snip_feature, seg_lens, topic_embedding, w_bb, b_bb, w_bt, b_bt, g1, g2):
    raise NotImplementedError("write your pallas kernel here")


# speedup vs baseline: 8.6998x; 8.6998x over previous
import jax
import jax.numpy as jnp
from jax.experimental import pallas as pl
from jax.experimental.pallas import tpu as pltpu

_B, _C, _T, _TD = 8, 256, 512, 16
_K = 3


def _relu(a):
    return jnp.maximum(a, 0.0)


def _dot(a, b):
    return jax.lax.dot_general(a, b, (((1,), (0,)), ((), ())),
                               preferred_element_type=jnp.float32)


def _shift_right(a):
    return jnp.concatenate([jnp.zeros_like(a[:, :1]), a[:, :-1]], axis=1)


def _shift_left(a):
    return jnp.concatenate([a[:, 1:], jnp.zeros_like(a[:, :1])], axis=1)


def _block_diag(w, groups):
    o, ig = w.shape
    og = o // groups
    r = w.reshape(groups, og, ig)
    eye = jnp.eye(groups, dtype=w.dtype)
    full = r[:, :, None, :] * eye[:, None, :, None]
    return full.reshape(o, groups * ig)


def _layer(x, tf_col, valid, p):
    t1 = _relu(_dot(p['t1'], x) + p['bt1'])
    t2 = _relu(_shift_right(_dot(p['t2a'], t1)) + _dot(p['t2b'], t1)
               + _shift_left(_dot(p['t2c'], t1)) + p['bt2'])
    tout = _relu(_dot(p['t3'], t2) + p['bt3'])

    g = jax.lax.dot_general(x, x, (((0,), (0,)), ((), ())),
                            preferred_element_type=jnp.float32)
    sq = jnp.sum(x * x, axis=0, keepdims=True)
    score = jnp.where(valid, sq - 2.0 * g, 1e9)
    lane = jax.lax.broadcasted_iota(jnp.int32, (_T, _T), 1)

    u = _dot(p['amd'], x) + p['sb1']
    v = _dot(p['d'], x)
    ve = _dot(p['d'], tf_col)

    def edge(s1):
        s2 = _relu(_dot(p['s2'], _relu(s1)) + p['sb2'])
        return _dot(p['s3'], s2) + p['sb3']

    m = edge(u + ve)
    for _ in range(_K):
        mn = jnp.min(score, axis=1, keepdims=True)
        idx = jnp.min(jnp.where(score == mn, lane, _T), axis=1, keepdims=True)
        oh = (lane == idx).astype(jnp.float32)
        score = jnp.where(lane == idx, 1e9, score)
        nbr = jax.lax.dot_general(v, oh, (((1,), (1,)), ((), ())),
                                  preferred_element_type=jnp.float32,
                                  precision=jax.lax.Precision.HIGHEST)
        m = jnp.maximum(m, edge(u + nbr))
    sout = _relu(m)
    return _relu(tout + x + sout)


def _body(x_ref, valid_ref, topic_ref,
          bb_a_ref, bb_b_ref, bb_c_ref, bbb_ref, wbt_ref, bbt_ref,
          g1t1, g1bt1, g1t2a, g1t2b, g1t2c, g1bt2, g1t3, g1bt3,
          g1amd, g1d, g1sb1, g1s2, g1sb2, g1s3, g1sb3,
          g2t1, g2bt1, g2t2a, g2t2b, g2t2c, g2bt2, g2t3, g2bt3,
          g2amd, g2d, g2sb1, g2s2, g2sb2, g2s3, g2sb3,
          out_ref):
    x0 = x_ref[0]
    valid = valid_ref[0] == 1.0

    base = _relu(_shift_right(_dot(bb_a_ref[...], x0))
                 + _dot(bb_b_ref[...], x0)
                 + _shift_left(_dot(bb_c_ref[...], x0)) + bbb_ref[...])

    trow = topic_ref[0]
    tf_col = _relu(jnp.sum(wbt_ref[...] * trow, axis=1, keepdims=True)
                   + bbt_ref[...])

    p1 = dict(t1=g1t1[...], bt1=g1bt1[...], t2a=g1t2a[...], t2b=g1t2b[...],
              t2c=g1t2c[...], bt2=g1bt2[...], t3=g1t3[...], bt3=g1bt3[...],
              amd=g1amd[...], d=g1d[...], sb1=g1sb1[...], s2=g1s2[...],
              sb2=g1sb2[...], s3=g1s3[...], sb3=g1sb3[...])
    p2 = dict(t1=g2t1[...], bt1=g2bt1[...], t2a=g2t2a[...], t2b=g2t2b[...],
              t2c=g2t2c[...], bt2=g2bt2[...], t3=g2t3[...], bt3=g2bt3[...],
              amd=g2amd[...], d=g2d[...], sb1=g2sb1[...], s2=g2s2[...],
              sb2=g2sb2[...], s3=g2s3[...], sb3=g2sb3[...])

    x1 = _layer(base, tf_col, valid, p1)
    out_ref[0] = _layer(x1, tf_col, valid, p2)


def _prep_gcn(g):
    col = lambda b: b[:, None]
    return [
        g['tw1'][:, :, 0], col(g['tb1']),
        _block_diag(g['tw2'][:, :, 0], 32), _block_diag(g['tw2'][:, :, 1], 32),
        _block_diag(g['tw2'][:, :, 2], 32), col(g['tb2']),
        g['tw3'][:, :, 0], col(g['tb3']),
        g['sw1'][:, :_C, 0, 0] - g['sw1'][:, _C:, 0, 0],
        g['sw1'][:, _C:, 0, 0],
        col(g['sb1']), _block_diag(g['sw2'][:, :, 0, 0], 32), col(g['sb2']),
        g['sw3'][:, :, 0, 0], col(g['sb3']),
    ]


def kernel(snip_feature, seg_lens, topic_embedding, w_bb, b_bb, w_bt, b_bt,
           g1, g2, interpret=False):
    seg = jnp.maximum(seg_lens, _K + 1).astype(jnp.int32)
    validf = (jnp.arange(_T, dtype=jnp.int32)[None, :]
              < seg[:, None]).astype(jnp.float32)[:, None, :]

    bb = [_block_diag(w_bb[:, :, j], 4) for j in range(3)]
    inputs = ([snip_feature, validf, topic_embedding[:, None, :]]
              + bb + [b_bb[:, None], _block_diag(w_bt[:, :, 0], 4),
                      b_bt[:, None]]
              + _prep_gcn(g1) + _prep_gcn(g2))

    rep = lambda a: pl.BlockSpec(a.shape, lambda b: (0,) * a.ndim)
    in_specs = [pl.BlockSpec((1, _C, _T), lambda b: (b, 0, 0)),
                pl.BlockSpec((1, 1, _T), lambda b: (b, 0, 0)),
                pl.BlockSpec((1, 1, _TD), lambda b: (b, 0, 0))]
    in_specs += [rep(a) for a in inputs[3:]]

    return pl.pallas_call(
        _body,
        grid=(_B,),
        in_specs=in_specs,
        out_specs=pl.BlockSpec((1, _C, _T), lambda b: (b, 0, 0)),
        out_shape=jax.ShapeDtypeStruct((_B, _C, _T), jnp.float32),
        interpret=interpret,
    )(*inputs)


# speedup vs baseline: 11.9969x; 1.3790x over previous
import jax
import jax.numpy as jnp
from jax.experimental import pallas as pl
from jax.experimental.pallas import tpu as pltpu

_B, _C, _T, _TD = 8, 256, 512, 16
_K = 3


def _relu(a):
    return jnp.maximum(a, 0.0)


def _dot(a, b):
    return jax.lax.dot_general(a, b, (((1,), (0,)), ((), ())),
                               preferred_element_type=jnp.float32)


def _shift_right(a):
    return jnp.concatenate([jnp.zeros_like(a[:, :1]), a[:, :-1]], axis=1)


def _shift_left(a):
    return jnp.concatenate([a[:, 1:], jnp.zeros_like(a[:, :1])], axis=1)


def _block_diag(w, groups):
    o, ig = w.shape
    og = o // groups
    r = w.reshape(groups, og, ig)
    eye = jnp.eye(groups, dtype=w.dtype)
    full = r[:, :, None, :] * eye[:, None, :, None]
    return full.reshape(o, groups * ig)


def _layer(x, tf_col, valid, p):
    t1 = _relu(_dot(p['t1'], x) + p['bt1'])
    t2 = _relu(_shift_right(_dot(p['t2a'], t1)) + _dot(p['t2b'], t1)
               + _shift_left(_dot(p['t2c'], t1)) + p['bt2'])
    tout = _relu(_dot(p['t3'], t2) + p['bt3'])

    g = jax.lax.dot_general(x, x, (((0,), (0,)), ((), ())),
                            preferred_element_type=jnp.float32)
    sq = jnp.sum(x * x, axis=0, keepdims=True)
    sq_col = jnp.swapaxes(sq, 0, 1)
    score = jnp.where(valid, sq_col - 2.0 * g, 1e9)
    sub = jax.lax.broadcasted_iota(jnp.int32, (_T, _T), 0)

    u = _dot(p['amd'], x) + p['sb1']
    v = _dot(p['d'], x)
    ve = _dot(p['d'], tf_col)

    def edge(s1):
        s2 = _relu(_dot(p['s2'], _relu(s1)) + p['sb2'])
        return _dot(p['s3'], s2) + p['sb3']

    m = edge(u + ve)
    for _ in range(_K):
        mn = jnp.min(score, axis=0, keepdims=True)
        idx = jnp.min(jnp.where(score == mn, sub, _T),
                      axis=0, keepdims=True)
        sel = sub == idx
        score = jnp.where(sel, 1e9, score)
        oh = sel.astype(jnp.float32)
        nbr = jax.lax.dot_general(v, oh, (((1,), (0,)), ((), ())),
                                  preferred_element_type=jnp.float32)
        m = jnp.maximum(m, edge(u + nbr))
    sout = _relu(m)
    return _relu(tout + x + sout)


def _body(x_ref, valid_ref, topic_ref,
          bb_a_ref, bb_b_ref, bb_c_ref, bbb_ref, wbt_ref, bbt_ref,
          g1t1, g1bt1, g1t2a, g1t2b, g1t2c, g1bt2, g1t3, g1bt3,
          g1amd, g1d, g1sb1, g1s2, g1sb2, g1s3, g1sb3,
          g2t1, g2bt1, g2t2a, g2t2b, g2t2c, g2bt2, g2t3, g2bt3,
          g2amd, g2d, g2sb1, g2s2, g2sb2, g2s3, g2sb3,
          out_ref):
    x0 = x_ref[0]
    valid = valid_ref[0] == 1.0

    base = _relu(_shift_right(_dot(bb_a_ref[...], x0))
                 + _dot(bb_b_ref[...], x0)
                 + _shift_left(_dot(bb_c_ref[...], x0)) + bbb_ref[...])

    trow = topic_ref[0]
    tf_col = _relu(jnp.sum(wbt_ref[...] * trow, axis=1, keepdims=True)
                   + bbt_ref[...])

    p1 = dict(t1=g1t1[...], bt1=g1bt1[...], t2a=g1t2a[...], t2b=g1t2b[...],
              t2c=g1t2c[...], bt2=g1bt2[...], t3=g1t3[...], bt3=g1bt3[...],
              amd=g1amd[...], d=g1d[...], sb1=g1sb1[...], s2=g1s2[...],
              sb2=g1sb2[...], s3=g1s3[...], sb3=g1sb3[...])
    p2 = dict(t1=g2t1[...], bt1=g2bt1[...], t2a=g2t2a[...], t2b=g2t2b[...],
              t2c=g2t2c[...], bt2=g2bt2[...], t3=g2t3[...], bt3=g2bt3[...],
              amd=g2amd[...], d=g2d[...], sb1=g2sb1[...], s2=g2s2[...],
              sb2=g2sb2[...], s3=g2s3[...], sb3=g2sb3[...])

    x1 = _layer(base, tf_col, valid, p1)
    out_ref[0] = _layer(x1, tf_col, valid, p2)


def _prep_gcn(g):
    col = lambda b: b[:, None]
    return [
        g['tw1'][:, :, 0], col(g['tb1']),
        _block_diag(g['tw2'][:, :, 0], 32), _block_diag(g['tw2'][:, :, 1], 32),
        _block_diag(g['tw2'][:, :, 2], 32), col(g['tb2']),
        g['tw3'][:, :, 0], col(g['tb3']),
        g['sw1'][:, :_C, 0, 0] - g['sw1'][:, _C:, 0, 0],
        g['sw1'][:, _C:, 0, 0],
        col(g['sb1']), _block_diag(g['sw2'][:, :, 0, 0], 32), col(g['sb2']),
        g['sw3'][:, :, 0, 0], col(g['sb3']),
    ]


def kernel(snip_feature, seg_lens, topic_embedding, w_bb, b_bb, w_bt, b_bt,
           g1, g2, interpret=False):
    seg = jnp.maximum(seg_lens, _K + 1).astype(jnp.int32)
    validf = (jnp.arange(_T, dtype=jnp.int32)[None, :]
              < seg[:, None]).astype(jnp.float32)[:, :, None]

    bb = [_block_diag(w_bb[:, :, j], 4) for j in range(3)]
    inputs = ([snip_feature, validf, topic_embedding[:, None, :]]
              + bb + [b_bb[:, None], _block_diag(w_bt[:, :, 0], 4),
                      b_bt[:, None]]
              + _prep_gcn(g1) + _prep_gcn(g2))

    rep = lambda a: pl.BlockSpec(a.shape, lambda b: (0,) * a.ndim)
    in_specs = [pl.BlockSpec((1, _C, _T), lambda b: (b, 0, 0)),
                pl.BlockSpec((1, _T, 1), lambda b: (b, 0, 0)),
                pl.BlockSpec((1, 1, _TD), lambda b: (b, 0, 0))]
    in_specs += [rep(a) for a in inputs[3:]]

    return pl.pallas_call(
        _body,
        grid=(_B,),
        in_specs=in_specs,
        out_specs=pl.BlockSpec((1, _C, _T), lambda b: (b, 0, 0)),
        out_shape=jax.ShapeDtypeStruct((_B, _C, _T), jnp.float32),
        interpret=interpret,
    )(*inputs)


# speedup vs baseline: 13.2499x; 1.1044x over previous
import jax
import jax.numpy as jnp
from jax.experimental import pallas as pl
from jax.experimental.pallas import tpu as pltpu

_B, _C, _T, _TD = 8, 256, 512, 16
_K = 3


def _relu(a):
    return jnp.maximum(a, 0.0)


def _dot(a, b):
    return jax.lax.dot_general(a, b, (((1,), (0,)), ((), ())),
                               preferred_element_type=jnp.float32)


def _shift_right(a):
    return jnp.concatenate([jnp.zeros_like(a[:, :1]), a[:, :-1]], axis=1)


def _shift_left(a):
    return jnp.concatenate([a[:, 1:], jnp.zeros_like(a[:, :1])], axis=1)


def _block_diag(w, groups):
    o, ig = w.shape
    og = o // groups
    r = w.reshape(groups, og, ig)
    eye = jnp.eye(groups, dtype=w.dtype)
    full = r[:, :, None, :] * eye[:, None, :, None]
    return full.reshape(o, groups * ig)


def _layer(x, tf_col, valid, p):
    t1 = _relu(_dot(p['t1'], x) + p['bt1'])
    t2 = _relu(_shift_right(_dot(p['t2a'], t1)) + _dot(p['t2b'], t1)
               + _shift_left(_dot(p['t2c'], t1)) + p['bt2'])
    tout = _relu(_dot(p['t3'], t2) + p['bt3'])

    g = jax.lax.dot_general(x, x, (((0,), (0,)), ((), ())),
                            preferred_element_type=jnp.float32)
    sq = jnp.sum(x * x, axis=0, keepdims=True)
    sq_col = jnp.swapaxes(sq, 0, 1)
    score = jnp.where(valid, sq_col - 2.0 * g, 1e9)
    sub = jax.lax.broadcasted_iota(jnp.int32, (_T, _T), 0)

    uv = _dot(p['uv'], x)
    u = uv[:128] + p['sb1']
    v = uv[128:]
    ve = _dot(p['d'], tf_col)

    s1 = [_relu(u + ve)]
    for j in range(_K):
        mn = jnp.min(score, axis=0, keepdims=True)
        idx = jnp.min(jnp.where(score == mn, sub, _T),
                      axis=0, keepdims=True)
        sel = sub == idx
        if j < _K - 1:
            score = jnp.where(sel, 1e9, score)
        oh = sel.astype(jnp.float32)
        nbr = jax.lax.dot_general(v, oh, (((1,), (0,)), ((), ())),
                                  preferred_element_type=jnp.float32)
        s1.append(_relu(u + nbr))
    s1 = jnp.concatenate(s1, axis=1)
    s2 = _relu(_dot(p['s2'], s1) + p['sb2'])
    s3 = _dot(p['s3'], s2) + p['sb3']
    m = jnp.maximum(jnp.maximum(s3[:, :_T], s3[:, _T:2 * _T]),
                    jnp.maximum(s3[:, 2 * _T:3 * _T], s3[:, 3 * _T:]))
    sout = _relu(m)
    return _relu(tout + x + sout)


def _body(x_ref, valid_ref, topic_ref,
          bb_a_ref, bb_b_ref, bb_c_ref, bbb_ref, wbt_ref, bbt_ref,
          g1t1, g1bt1, g1t2a, g1t2b, g1t2c, g1bt2, g1t3, g1bt3,
          g1amd, g1d, g1sb1, g1s2, g1sb2, g1s3, g1sb3,
          g2t1, g2bt1, g2t2a, g2t2b, g2t2c, g2bt2, g2t3, g2bt3,
          g2amd, g2d, g2sb1, g2s2, g2sb2, g2s3, g2sb3,
          out_ref):
    x0 = x_ref[0]
    valid = valid_ref[0] == 1.0

    base = _relu(_shift_right(_dot(bb_a_ref[...], x0))
                 + _dot(bb_b_ref[...], x0)
                 + _shift_left(_dot(bb_c_ref[...], x0)) + bbb_ref[...])

    trow = topic_ref[0]
    tf_col = _relu(jnp.sum(wbt_ref[...] * trow, axis=1, keepdims=True)
                   + bbt_ref[...])

    p1 = dict(t1=g1t1[...], bt1=g1bt1[...], t2a=g1t2a[...], t2b=g1t2b[...],
              t2c=g1t2c[...], bt2=g1bt2[...], t3=g1t3[...], bt3=g1bt3[...],
              uv=g1amd[...], d=g1d[...], sb1=g1sb1[...], s2=g1s2[...],
              sb2=g1sb2[...], s3=g1s3[...], sb3=g1sb3[...])
    p2 = dict(t1=g2t1[...], bt1=g2bt1[...], t2a=g2t2a[...], t2b=g2t2b[...],
              t2c=g2t2c[...], bt2=g2bt2[...], t3=g2t3[...], bt3=g2bt3[...],
              uv=g2amd[...], d=g2d[...], sb1=g2sb1[...], s2=g2s2[...],
              sb2=g2sb2[...], s3=g2s3[...], sb3=g2sb3[...])

    x1 = _layer(base, tf_col, valid, p1)
    out_ref[0] = _layer(x1, tf_col, valid, p2)


def _prep_gcn(g):
    col = lambda b: b[:, None]
    return [
        g['tw1'][:, :, 0], col(g['tb1']),
        _block_diag(g['tw2'][:, :, 0], 32), _block_diag(g['tw2'][:, :, 1], 32),
        _block_diag(g['tw2'][:, :, 2], 32), col(g['tb2']),
        g['tw3'][:, :, 0], col(g['tb3']),
        jnp.concatenate([g['sw1'][:, :_C, 0, 0] - g['sw1'][:, _C:, 0, 0],
                         g['sw1'][:, _C:, 0, 0]], axis=0),
        g['sw1'][:, _C:, 0, 0],
        col(g['sb1']), _block_diag(g['sw2'][:, :, 0, 0], 32), col(g['sb2']),
        g['sw3'][:, :, 0, 0], col(g['sb3']),
    ]


def kernel(snip_feature, seg_lens, topic_embedding, w_bb, b_bb, w_bt, b_bt,
           g1, g2, interpret=False):
    seg = jnp.maximum(seg_lens, _K + 1).astype(jnp.int32)
    validf = (jnp.arange(_T, dtype=jnp.int32)[None, :]
              < seg[:, None]).astype(jnp.float32)[:, :, None]

    bb = [_block_diag(w_bb[:, :, j], 4) for j in range(3)]
    inputs = ([snip_feature, validf, topic_embedding[:, None, :]]
              + bb + [b_bb[:, None], _block_diag(w_bt[:, :, 0], 4),
                      b_bt[:, None]]
              + _prep_gcn(g1) + _prep_gcn(g2))

    rep = lambda a: pl.BlockSpec(a.shape, lambda b: (0,) * a.ndim)
    in_specs = [pl.BlockSpec((1, _C, _T), lambda b: (b, 0, 0)),
                pl.BlockSpec((1, _T, 1), lambda b: (b, 0, 0)),
                pl.BlockSpec((1, 1, _TD), lambda b: (b, 0, 0))]
    in_specs += [rep(a) for a in inputs[3:]]

    return pl.pallas_call(
        _body,
        grid=(_B,),
        in_specs=in_specs,
        out_specs=pl.BlockSpec((1, _C, _T), lambda b: (b, 0, 0)),
        out_shape=jax.ShapeDtypeStruct((_B, _C, _T), jnp.float32),
        interpret=interpret,
    )(*inputs)


# speedup vs baseline: 15.8479x; 1.1961x over previous
import jax
import jax.numpy as jnp
from jax.experimental import pallas as pl
from jax.experimental.pallas import tpu as pltpu

_B, _C, _T, _TD = 8, 256, 512, 16
_K = 3


def _relu(a):
    return jnp.maximum(a, 0.0)


def _dot(a, b):
    return jax.lax.dot_general(a, b, (((1,), (0,)), ((), ())),
                               preferred_element_type=jnp.float32)


def _shift_right(a):
    return jnp.concatenate([jnp.zeros_like(a[:, :1]), a[:, :-1]], axis=1)


def _shift_left(a):
    return jnp.concatenate([a[:, 1:], jnp.zeros_like(a[:, :1])], axis=1)


def _bd_expand(w, groups, out_lanes_per_group):
    o, ig = w.shape
    og = o // groups
    tiled = jnp.concatenate([w] * groups, axis=1)
    sub = jax.lax.broadcasted_iota(jnp.int32, (o, groups * ig), 0)
    lane = jax.lax.broadcasted_iota(jnp.int32, (o, groups * ig), 1)
    return jnp.where((lane // ig) == (sub // og), tiled, 0.0)


def _layer(x, tf_col, valid, p):
    t1 = _relu(_dot(p['t1'], x) + p['bt1'])
    t2 = _relu(_shift_right(_dot(p['t2s'][0], t1)) + _dot(p['t2s'][1], t1)
               + _shift_left(_dot(p['t2s'][2], t1)) + p['bt2'])
    tout = _relu(_dot(p['t3'], t2) + p['bt3'])

    g = jax.lax.dot_general(x, x, (((0,), (0,)), ((), ())),
                            preferred_element_type=jnp.float32)
    sq = jnp.sum(x * x, axis=0, keepdims=True)
    sq_col = jnp.swapaxes(sq, 0, 1)
    score = jnp.where(valid, sq_col - 2.0 * g, 1e9)
    sub = jax.lax.broadcasted_iota(jnp.int32, (_T, _T), 0)

    uv = _dot(p['uv'], x)
    u = uv[:128] + p['sb1']
    v = uv[128:]
    ve = _dot(p['uv'][128:], tf_col)

    s1 = [_relu(u + ve)]
    for j in range(_K):
        mn = jnp.min(score, axis=0, keepdims=True)
        idx = jnp.min(jnp.where(score == mn, sub, _T),
                      axis=0, keepdims=True)
        sel = sub == idx
        if j < _K - 1:
            score = jnp.where(sel, 1e9, score)
        oh = sel.astype(jnp.float32)
        nbr = jax.lax.dot_general(v, oh, (((1,), (0,)), ((), ())),
                                  preferred_element_type=jnp.float32)
        s1.append(_relu(u + nbr))
    s1 = jnp.concatenate(s1, axis=1)
    s2 = _relu(_dot(p['s2'], s1) + p['sb2'])
    s3 = _dot(p['s3'], s2) + p['sb3']
    m = jnp.maximum(jnp.maximum(s3[:, :_T], s3[:, _T:2 * _T]),
                    jnp.maximum(s3[:, 2 * _T:3 * _T], s3[:, 3 * _T:]))
    sout = _relu(m)
    return _relu(tout + x + sout)


def _body(x_ref, seg_ref, topic_ref, wbb_ref, bbb_ref, wbt_ref, bbt_ref,
          g1t1, g1bt1, g1t2, g1bt2, g1t3, g1bt3, g1s1, g1sb1, g1s2, g1sb2,
          g1s3, g1sb3,
          g2t1, g2bt1, g2t2, g2bt2, g2t3, g2bt3, g2s1, g2sb1, g2s2, g2sb2,
          g2s3, g2sb3,
          out_ref,
          bb_s, wbt_s, t2a_s, s2a_s, uva_s, t2b_s, s2b_s, uvb_s):
    b = pl.program_id(0)

    @pl.when(b == 0)
    def _prep():
        for j in range(3):
            bb_s[j] = _bd_expand(wbb_ref[j], 4, 64)
            t2a_s[j] = _bd_expand(g1t2[j], 32, 4)
            t2b_s[j] = _bd_expand(g2t2[j], 32, 4)
        wbt_s[...] = _bd_expand(wbt_ref[...], 4, 4)
        s2a_s[...] = _bd_expand(g1s2[...], 32, 4)
        s2b_s[...] = _bd_expand(g2s2[...], 32, 4)
        w1a = g1s1[...]
        uva_s[...] = jnp.concatenate(
            [w1a[:, :_C] - w1a[:, _C:], w1a[:, _C:]], axis=0)
        w1b = g2s1[...]
        uvb_s[...] = jnp.concatenate(
            [w1b[:, :_C] - w1b[:, _C:], w1b[:, _C:]], axis=0)

    x0 = x_ref[0]
    seg = jnp.maximum(seg_ref[b], _K + 1)
    valid = jax.lax.broadcasted_iota(jnp.int32, (_T, 1), 0) < seg

    base = _relu(_shift_right(_dot(bb_s[0], x0))
                 + _dot(bb_s[1], x0)
                 + _shift_left(_dot(bb_s[2], x0)) + bbb_ref[...])

    trow = topic_ref[0]
    tf_col = _relu(jnp.sum(wbt_s[...] * trow, axis=1, keepdims=True)
                   + bbt_ref[...])

    p1 = dict(t1=g1t1[...], bt1=g1bt1[...], t2s=t2a_s, bt2=g1bt2[...],
              t3=g1t3[...], bt3=g1bt3[...], uv=uva_s[...], sb1=g1sb1[...],
              s2=s2a_s[...], sb2=g1sb2[...], s3=g1s3[...], sb3=g1sb3[...])
    p2 = dict(t1=g2t1[...], bt1=g2bt1[...], t2s=t2b_s, bt2=g2bt2[...],
              t3=g2t3[...], bt3=g2bt3[...], uv=uvb_s[...], sb1=g2sb1[...],
              s2=s2b_s[...], sb2=g2sb2[...], s3=g2s3[...], sb3=g2sb3[...])

    x1 = _layer(base, tf_col, valid, p1)
    out_ref[0] = _layer(x1, tf_col, valid, p2)


def _prep_gcn(g):
    col = lambda b: b[:, None]
    return [
        g['tw1'][:, :, 0], col(g['tb1']),
        jnp.transpose(g['tw2'], (2, 0, 1)), col(g['tb2']),
        g['tw3'][:, :, 0], col(g['tb3']),
        g['sw1'][:, :, 0, 0], col(g['sb1']),
        g['sw2'][:, :, 0, 0], col(g['sb2']),
        g['sw3'][:, :, 0, 0], col(g['sb3']),
    ]


def kernel(snip_feature, seg_lens, topic_embedding, w_bb, b_bb, w_bt, b_bt,
           g1, g2, interpret=False):
    inputs = ([snip_feature, seg_lens.astype(jnp.int32),
               topic_embedding[:, None, :],
               jnp.transpose(w_bb, (2, 0, 1)), b_bb[:, None],
               w_bt[:, :, 0], b_bt[:, None]]
              + _prep_gcn(g1) + _prep_gcn(g2))

    rep = lambda a: pl.BlockSpec(a.shape, lambda b: (0,) * a.ndim)
    in_specs = [pl.BlockSpec((1, _C, _T), lambda b: (b, 0, 0)),
                pl.BlockSpec(memory_space=pltpu.SMEM),
                pl.BlockSpec((1, 1, _TD), lambda b: (b, 0, 0))]
    in_specs += [rep(a) for a in inputs[3:]]

    scratch = [pltpu.VMEM((3, _C, _C), jnp.float32),
               pltpu.VMEM((_C, _TD), jnp.float32),
               pltpu.VMEM((3, 128, 128), jnp.float32),
               pltpu.VMEM((128, 128), jnp.float32),
               pltpu.VMEM((_C, _C), jnp.float32),
               pltpu.VMEM((3, 128, 128), jnp.float32),
               pltpu.VMEM((128, 128), jnp.float32),
               pltpu.VMEM((_C, _C), jnp.float32)]

    return pl.pallas_call(
        _body,
        grid=(_B,),
        in_specs=in_specs,
        out_specs=pl.BlockSpec((1, _C, _T), lambda b: (b, 0, 0)),
        out_shape=jax.ShapeDtypeStruct((_B, _C, _T), jnp.float32),
        scratch_shapes=scratch,
        interpret=interpret,
    )(*inputs)


# speedup vs baseline: 16.9694x; 1.0708x over previous
import jax
import jax.numpy as jnp
from jax.experimental import pallas as pl
from jax.experimental.pallas import tpu as pltpu

_B, _C, _T, _TD = 8, 256, 512, 16
_K = 3


def _relu(a):
    return jnp.maximum(a, 0.0)


def _dot(a, b):
    return jax.lax.dot_general(a, b, (((1,), (0,)), ((), ())),
                               preferred_element_type=jnp.float32)


def _shift_right(a):
    return jnp.concatenate([jnp.zeros_like(a[:, :1]), a[:, :-1]], axis=1)


def _shift_left(a):
    return jnp.concatenate([a[:, 1:], jnp.zeros_like(a[:, :1])], axis=1)


def _bd_expand(w, groups):
    o, ig = w.shape
    og = o // groups
    n = groups * ig
    tsub = jax.lax.broadcasted_iota(jnp.int32, (ig, n), 0)
    tlane = jax.lax.broadcasted_iota(jnp.int32, (ig, n), 1)
    tile_mat = (tlane % ig == tsub).astype(jnp.float32)
    tiled = _dot(w, tile_mat)
    sub = jax.lax.broadcasted_iota(jnp.int32, (o, n), 0)
    lane = jax.lax.broadcasted_iota(jnp.int32, (o, n), 1)
    return jnp.where((lane // ig) == (sub // og), tiled, 0.0)


def _layer(x, tf_col, valid, p):
    t1 = _relu(_dot(p['t1'], x) + p['bt1'])
    t2 = _relu(_shift_right(_dot(p['t2s'][0], t1)) + _dot(p['t2s'][1], t1)
               + _shift_left(_dot(p['t2s'][2], t1)) + p['bt2'])
    tout = _relu(_dot(p['t3'], t2) + p['bt3'])

    g = jax.lax.dot_general(x, x, (((0,), (0,)), ((), ())),
                            preferred_element_type=jnp.float32)
    sq = jnp.sum(x * x, axis=0, keepdims=True)
    sq_col = jnp.swapaxes(sq, 0, 1)
    score = jnp.where(valid, sq_col - 2.0 * g, 1e9)
    sub = jax.lax.broadcasted_iota(jnp.int32, (_T, _T), 0)

    uv = _dot(p['uv'], x)
    u = uv[:128] + p['sb1']
    v = uv[128:]
    ve = _dot(p['uv'][128:], tf_col)

    s1 = [_relu(u + ve)]
    for j in range(_K):
        mn = jnp.min(score, axis=0, keepdims=True)
        idx = jnp.min(jnp.where(score == mn, sub, _T),
                      axis=0, keepdims=True)
        sel = sub == idx
        if j < _K - 1:
            score = jnp.where(sel, 1e9, score)
        oh = sel.astype(jnp.float32)
        nbr = jax.lax.dot_general(v, oh, (((1,), (0,)), ((), ())),
                                  preferred_element_type=jnp.float32)
        s1.append(_relu(u + nbr))
    s1 = jnp.concatenate(s1, axis=1)
    s2 = _relu(_dot(p['s2'], s1) + p['sb2'])
    s3 = _dot(p['s3'], s2) + p['sb3']
    m = jnp.maximum(jnp.maximum(s3[:, :_T], s3[:, _T:2 * _T]),
                    jnp.maximum(s3[:, 2 * _T:3 * _T], s3[:, 3 * _T:]))
    sout = _relu(m)
    return _relu(tout + x + sout)


def _body(x_ref, seg_ref, topic_ref, wbb_ref, bbb_ref, wbt_ref, bbt_ref,
          g1t1, g1bt1, g1t2, g1bt2, g1t3, g1bt3, g1s1, g1sb1, g1s2, g1sb2,
          g1s3, g1sb3,
          g2t1, g2bt1, g2t2, g2bt2, g2t3, g2bt3, g2s1, g2sb1, g2s2, g2sb2,
          g2s3, g2sb3,
          out_ref,
          bb_s, wbt_s, t2a_s, s2a_s, uva_s, t2b_s, s2b_s, uvb_s):
    b = pl.program_id(0)

    @pl.when(b == 0)
    def _prep():
        for j in range(3):
            bb_s[j] = _bd_expand(wbb_ref[j], 4)
            t2a_s[j] = _bd_expand(g1t2[j], 32)
            t2b_s[j] = _bd_expand(g2t2[j], 32)
        wbt_s[...] = _bd_expand(wbt_ref[...], 4)
        s2a_s[...] = _bd_expand(g1s2[...], 32)
        s2b_s[...] = _bd_expand(g2s2[...], 32)
        w1a = g1s1[...]
        uva_s[...] = jnp.concatenate(
            [w1a[:, :_C] - w1a[:, _C:], w1a[:, _C:]], axis=0)
        w1b = g2s1[...]
        uvb_s[...] = jnp.concatenate(
            [w1b[:, :_C] - w1b[:, _C:], w1b[:, _C:]], axis=0)

    x0 = x_ref[0]
    seg = jnp.maximum(seg_ref[b], _K + 1)
    valid = jax.lax.broadcasted_iota(jnp.int32, (_T, 1), 0) < seg

    base = _relu(_shift_right(_dot(bb_s[0], x0))
                 + _dot(bb_s[1], x0)
                 + _shift_left(_dot(bb_s[2], x0)) + bbb_ref[...])

    trow = topic_ref[0]
    tf_col = _relu(jnp.sum(wbt_s[...] * trow, axis=1, keepdims=True)
                   + bbt_ref[...])

    p1 = dict(t1=g1t1[...], bt1=g1bt1[...], t2s=t2a_s, bt2=g1bt2[...],
              t3=g1t3[...], bt3=g1bt3[...], uv=uva_s[...], sb1=g1sb1[...],
              s2=s2a_s[...], sb2=g1sb2[...], s3=g1s3[...], sb3=g1sb3[...])
    p2 = dict(t1=g2t1[...], bt1=g2bt1[...], t2s=t2b_s, bt2=g2bt2[...],
              t3=g2t3[...], bt3=g2bt3[...], uv=uvb_s[...], sb1=g2sb1[...],
              s2=s2b_s[...], sb2=g2sb2[...], s3=g2s3[...], sb3=g2sb3[...])

    x1 = _layer(base, tf_col, valid, p1)
    out_ref[0] = _layer(x1, tf_col, valid, p2)


def _prep_gcn(g):
    col = lambda b: b[:, None]
    return [
        g['tw1'][:, :, 0], col(g['tb1']),
        jnp.transpose(g['tw2'], (2, 0, 1)), col(g['tb2']),
        g['tw3'][:, :, 0], col(g['tb3']),
        g['sw1'][:, :, 0, 0], col(g['sb1']),
        g['sw2'][:, :, 0, 0], col(g['sb2']),
        g['sw3'][:, :, 0, 0], col(g['sb3']),
    ]


def kernel(snip_feature, seg_lens, topic_embedding, w_bb, b_bb, w_bt, b_bt,
           g1, g2, interpret=False):
    inputs = ([snip_feature, seg_lens.astype(jnp.int32),
               topic_embedding[:, None, :],
               jnp.transpose(w_bb, (2, 0, 1)), b_bb[:, None],
               w_bt[:, :, 0], b_bt[:, None]]
              + _prep_gcn(g1) + _prep_gcn(g2))

    rep = lambda a: pl.BlockSpec(a.shape, lambda b: (0,) * a.ndim)
    in_specs = [pl.BlockSpec((1, _C, _T), lambda b: (b, 0, 0)),
                pl.BlockSpec(memory_space=pltpu.SMEM),
                pl.BlockSpec((1, 1, _TD), lambda b: (b, 0, 0))]
    in_specs += [rep(a) for a in inputs[3:]]

    scratch = [pltpu.VMEM((3, _C, _C), jnp.float32),
               pltpu.VMEM((_C, _TD), jnp.float32),
               pltpu.VMEM((3, 128, 128), jnp.float32),
               pltpu.VMEM((128, 128), jnp.float32),
               pltpu.VMEM((_C, _C), jnp.float32),
               pltpu.VMEM((3, 128, 128), jnp.float32),
               pltpu.VMEM((128, 128), jnp.float32),
               pltpu.VMEM((_C, _C), jnp.float32)]

    return pl.pallas_call(
        _body,
        grid=(_B,),
        in_specs=in_specs,
        out_specs=pl.BlockSpec((1, _C, _T), lambda b: (b, 0, 0)),
        out_shape=jax.ShapeDtypeStruct((_B, _C, _T), jnp.float32),
        scratch_shapes=scratch,
        interpret=interpret,
    )(*inputs)
